# baseline (speedup 1.0000x reference)
import functools

import jax
import jax.numpy as jnp
from jax import lax
from jax.experimental import pallas as pl
from jax.experimental.pallas import tpu as pltpu
from jax.experimental.pallas import tpu_sc as plsc

N = 10000
E = 10000
NNZ = 320000
D = 128
HALF = 64
NP = 10240
NC = 2
NS = 16
CH = 64
NCH = 320
NNZ_PAD = NS * NCH * CH
NPAD_BINS = 240
SLICE = NP // NS
SUB = 64
NBUF = 5
assert NCH % NBUF == 0
MM_BLK = 1280


def _matmul_body(x_ref, w_ref, o_ref):
    xw = jnp.dot(x_ref[...], w_ref[...], preferred_element_type=jnp.float32)
    o_ref[0] = xw[:, :HALF]
    o_ref[1] = xw[:, HALF:]


def _matmul(x_pad, W):
    return pl.pallas_call(
        _matmul_body,
        grid=(NP // MM_BLK,),
        in_specs=[
            pl.BlockSpec((MM_BLK, D), lambda i: (i, 0)),
            pl.BlockSpec((D, D), lambda i: (0, 0)),
        ],
        out_specs=pl.BlockSpec((NC, MM_BLK, HALF), lambda i: (0, i, 0)),
        out_shape=jax.ShapeDtypeStruct((NC, NP, HALF), jnp.float32),
    )(x_pad, W)


def _bn_body(a_ref, dd_ref, g_ref, be_ref, o_ref):
    dv = dd_ref[:N, :]
    dinv = jnp.where(dv > 0.0, 1.0 / dv, 0.0)
    for p in range(NC):
        h = a_ref[p, :N, :] * dinv
        mu = jnp.mean(h, axis=0, keepdims=True)
        var = jnp.mean(jnp.square(h - mu), axis=0, keepdims=True)
        g = g_ref[:, p * HALF:(p + 1) * HALF]
        be = be_ref[:, p * HALF:(p + 1) * HALF]
        o_ref[:, p * HALF:(p + 1) * HALF] = (
            g * (h - mu) * lax.rsqrt(var + 1e-5) + be)


def _batchnorm(out_r, dd, gamma, beta):
    return pl.pallas_call(
        _bn_body,
        out_shape=jax.ShapeDtypeStruct((N, D), jnp.float32),
    )(out_r, dd, gamma.reshape(1, D), beta.reshape(1, D))


def _sc_body(xwcat_hbm, pidx_hbm, out_hbm, ddeg_hbm,
             pidx_v, rows_v, big_v, deg_v, ones_v, idxg_v, idxs_v, idxn_v,
             agg, nagg, bdeg, ddeg, gsem, ssem, h1sem, h2sem):
    c = lax.axis_index("c")
    s = lax.axis_index("s")
    coff = c * NP
    zero16 = jnp.zeros((16,), jnp.float32)

    pltpu.sync_copy(pidx_hbm.at[s], pidx_v)
    pltpu.sync_copy(xwcat_hbm.at[pl.ds(coff + s * SLICE, SLICE)],
                    nagg.at[pl.ds(s * SLICE, SLICE)])

    def _zero_big(r, carry):
        for q in range(HALF // 16):
            big_v[r, pl.ds(q * 16, 16)] = zero16
        return carry

    def _zero_acc(accref):
        def _z(u, carry):
            pltpu.sync_copy(big_v, accref.at[pl.ds(s * SLICE + u * SUB, SUB)])
            return carry
        lax.fori_loop(0, SLICE // SUB, _z, 0)

    lax.fori_loop(0, SUB, _zero_big, 0)
    for q in range(SLICE // 16):
        deg_v[pl.ds(q * 16, 16)] = zero16
    for q in range(CH // 16):
        ones_v[pl.ds(q * 16, 16)] = jnp.ones((16,), jnp.float32)
    _zero_acc(agg)
    pltpu.sync_copy(deg_v, bdeg.at[pl.ds(s * SLICE, SLICE)])
    pltpu.sync_copy(deg_v, ddeg.at[pl.ds(s * SLICE, SLICE)])
    plsc.subcore_barrier()

    def _stream_pass(src, dst, gather_low):
        def _p(t, carry):
            gds, sds, hds = [], [], []
            for b in range(NBUF):
                j = t * NBUF + b
                for q in range(CH // 16):
                    v = pidx_v[j, pl.ds(q * 16, 16)]
                    lo = v & 0xFFFF
                    hi = lax.shift_right_logical(v, 16)
                    g = lo if gather_low else hi
                    sc = hi if gather_low else lo
                    idxg_v[b, pl.ds(q * 16, 16)] = g
                    idxs_v[b, pl.ds(q * 16, 16)] = sc
                    if gather_low:
                        idxn_v[b, pl.ds(q * 16, 16)] = lo
                gds.append(pltpu.async_copy(
                    src.at[idxg_v.at[b]], rows_v.at[b], gsem.at[b]))
                if gather_low:
                    hds.append((
                        pltpu.async_copy(ones_v, bdeg.at[idxs_v.at[b]],
                                         h1sem.at[b], add=True),
                        pltpu.async_copy(ones_v, ddeg.at[idxn_v.at[b]],
                                         h2sem.at[b], add=True)))
            for b in range(NBUF):
                gds[b].wait()
                sds.append(pltpu.async_copy(
                    rows_v.at[b], dst.at[idxs_v.at[b]], ssem.at[b],
                    add=True))
            for b in range(NBUF):
                sds[b].wait()
                if gather_low:
                    hds[b][0].wait()
                    hds[b][1].wait()
            return carry
        lax.fori_loop(0, NCH // NBUF, _p, 0)

    def _scale_in_place(degref, accref):
        pltpu.sync_copy(degref.at[pl.ds(s * SLICE, SLICE)], deg_v)

        def _inv(k, carry):
            v = deg_v[pl.ds(k * 16, 16)]
            deg_v[pl.ds(k * 16, 16)] = jnp.where(v > 0.0, 1.0 / v, 0.0)
            return carry
        lax.fori_loop(0, SLICE // 16, _inv, 0)

        def _u(u, ucarry):
            base = s * SLICE + u * SUB
            pltpu.sync_copy(accref.at[pl.ds(base, SUB)], big_v)

            def _row16(k, carry):
                vinv = deg_v[pl.ds(u * SUB + k * 16, 16)]
                for i in range(16):
                    r = k * 16 + i
                    inv = vinv[i]
                    for q in range(HALF // 16):
                        big_v[r, pl.ds(q * 16, 16)] = (
                            big_v[r, pl.ds(q * 16, 16)] * inv)
                return carry
            lax.fori_loop(0, SUB // 16, _row16, 0)
            pltpu.sync_copy(big_v, accref.at[pl.ds(base, SUB)])
            return ucarry
        lax.fori_loop(0, SLICE // SUB, _u, 0)

    _stream_pass(nagg, agg, gather_low=True)
    plsc.subcore_barrier()

    _scale_in_place(bdeg, agg)
    lax.fori_loop(0, SUB, _zero_big, 0)
    _zero_acc(nagg)
    plsc.subcore_barrier()

    _stream_pass(agg, nagg, gather_low=False)
    plsc.subcore_barrier()

    pltpu.sync_copy(nagg.at[pl.ds(s * SLICE, SLICE)],
                    out_hbm.at[pl.ds(coff + s * SLICE, SLICE)])
    pltpu.sync_copy(ddeg.at[pl.ds(s * SLICE, SLICE)],
                    ddeg_hbm.at[pl.ds(coff + s * SLICE, SLICE)])


def _build_sc_kernel(agg_rows):
    return functools.partial(
        pl.kernel,
        out_type=(
            jax.ShapeDtypeStruct((NC * NP, HALF), jnp.float32),
            jax.ShapeDtypeStruct((NC * NP,), jnp.float32),
        ),
        mesh=plsc.VectorSubcoreMesh(
            core_axis_name="c", subcore_axis_name="s",
            num_cores=NC, num_subcores=NS),
        compiler_params=pltpu.CompilerParams(use_tc_tiling_on_sc=False),
        scratch_types=[
            pltpu.VMEM((NCH, CH), jnp.int32),
            pltpu.VMEM((NBUF, CH, HALF), jnp.float32),
            pltpu.VMEM((SUB, HALF), jnp.float32),
            pltpu.VMEM((SLICE,), jnp.float32),
            pltpu.VMEM((CH,), jnp.float32),
            pltpu.VMEM((NBUF, CH), jnp.int32),
            pltpu.VMEM((NBUF, CH), jnp.int32),
            pltpu.VMEM((NBUF, CH), jnp.int32),
            pltpu.VMEM_SHARED((agg_rows, HALF), jnp.float32),
            pltpu.VMEM_SHARED((agg_rows, HALF), jnp.float32),
            pltpu.VMEM_SHARED((NP,), jnp.float32),
            pltpu.VMEM_SHARED((NP,), jnp.float32),
            pltpu.SemaphoreType.DMA((NBUF,)),
            pltpu.SemaphoreType.DMA((NBUF,)),
            pltpu.SemaphoreType.DMA((NBUF,)),
            pltpu.SemaphoreType.DMA((NBUF,)),
        ],
    )(_sc_body)


_sc_kernel = _build_sc_kernel(NP)


def kernel(x, hyperedge_index, hyperedge_attr, W, b, gamma, beta):
    del hyperedge_attr
    del b
    node_idx = hyperedge_index[0]
    edge_idx = hyperedge_index[1]

    x_pad = jnp.pad(x, ((0, NP - N), (0, 0)))
    pad_bins = (N + (jnp.arange(NNZ_PAD - NNZ, dtype=jnp.int32)
                     % NPAD_BINS)).astype(jnp.int32)
    nidx_p = jnp.concatenate([node_idx, pad_bins])
    eidx_p = jnp.concatenate([edge_idx, pad_bins])
    pidx = (nidx_p | (eidx_p << 16)).reshape(NS, NCH, CH)

    xw_cat = _matmul(x_pad, W).reshape(NC * NP, HALF)
    out_r, dd = _sc_kernel(xw_cat, pidx)
    return _batchnorm(out_r.reshape(NC, NP, HALF),
                      dd[:NP].reshape(NP, 1), gamma, beta)


# speedup vs baseline: 1.1134x; 1.1134x over previous
import functools

import jax
import jax.numpy as jnp
from jax import lax
from jax.experimental import pallas as pl
from jax.experimental.pallas import tpu as pltpu
from jax.experimental.pallas import tpu_sc as plsc

N = 10000
E = 10000
NNZ = 320000
D = 128
HALF = 64
NP = 10240
NC = 2
NS = 16
CH = 64
NCH = 320
NNZ_PAD = NS * NCH * CH
NPAD_BINS = 240
SLICE = NP // NS
SUB = 64
NBUF = 5
assert NCH % NBUF == 0
MM_BLK = 1280


def _matmul_body(x_ref, w_ref, o_ref):
    xw = jnp.dot(x_ref[...], w_ref[...], preferred_element_type=jnp.float32)
    o_ref[0] = xw[:, :HALF]
    o_ref[1] = xw[:, HALF:]


def _matmul(x_pad, W):
    return pl.pallas_call(
        _matmul_body,
        grid=(NP // MM_BLK,),
        in_specs=[
            pl.BlockSpec((MM_BLK, D), lambda i: (i, 0)),
            pl.BlockSpec((D, D), lambda i: (0, 0)),
        ],
        out_specs=pl.BlockSpec((NC, MM_BLK, HALF), lambda i: (0, i, 0)),
        out_shape=jax.ShapeDtypeStruct((NC, NP, HALF), jnp.float32),
    )(x_pad, W)


def _bn_body(a_ref, dd_ref, g_ref, be_ref, o_ref):
    dv = dd_ref[:N, :]
    dinv = jnp.where(dv > 0.0, 1.0 / dv, 0.0)
    for p in range(NC):
        h = a_ref[p, :N, :] * dinv
        mu = jnp.mean(h, axis=0, keepdims=True)
        var = jnp.mean(jnp.square(h - mu), axis=0, keepdims=True)
        g = g_ref[:, p * HALF:(p + 1) * HALF]
        be = be_ref[:, p * HALF:(p + 1) * HALF]
        o_ref[:, p * HALF:(p + 1) * HALF] = (
            g * (h - mu) * lax.rsqrt(var + 1e-5) + be)


def _batchnorm(out_r, dd, gamma, beta):
    return pl.pallas_call(
        _bn_body,
        out_shape=jax.ShapeDtypeStruct((N, D), jnp.float32),
    )(out_r, dd, gamma.reshape(1, D), beta.reshape(1, D))


def _sc_body(xwcat_hbm, pidx_hbm, out_hbm, ddeg_hbm,
             pidx_v, rows_v, big_v, deg_v, ones_v, idxg_v, idxs_v, idxn_v,
             agg, nagg, bdeg, ddeg, gsem, ssem, h1sem, h2sem):
    c = lax.axis_index("c")
    s = lax.axis_index("s")
    coff = c * NP
    zero16 = jnp.zeros((16,), jnp.float32)

    pltpu.sync_copy(pidx_hbm.at[s], pidx_v)

    def _zero_big(r, carry):
        for q in range(HALF // 16):
            big_v[r, pl.ds(q * 16, 16)] = zero16
        return carry

    def _zero_acc(accref):
        def _z(u, carry):
            pltpu.sync_copy(big_v, accref.at[pl.ds(s * SLICE + u * SUB, SUB)])
            return carry
        lax.fori_loop(0, SLICE // SUB, _z, 0)

    lax.fori_loop(0, SUB, _zero_big, 0)
    for q in range(SLICE // 16):
        deg_v[pl.ds(q * 16, 16)] = zero16
    for q in range(CH // 16):
        ones_v[pl.ds(q * 16, 16)] = jnp.ones((16,), jnp.float32)
    _zero_acc(agg)
    _zero_acc(nagg)
    pltpu.sync_copy(deg_v, bdeg.at[pl.ds(s * SLICE, SLICE)])
    pltpu.sync_copy(deg_v, ddeg.at[pl.ds(s * SLICE, SLICE)])
    plsc.subcore_barrier()

    def _stream_pass(src, dst, gather_low):
        def _p(t, carry):
            gds, sds, hds = [], [], []
            for b in range(NBUF):
                j = t * NBUF + b
                for q in range(CH // 16):
                    v = pidx_v[j, pl.ds(q * 16, 16)]
                    lo = v & 0xFFFF
                    hi = lax.shift_right_logical(v, 16)
                    g = (lo + coff) if gather_low else hi
                    sc = hi if gather_low else lo
                    idxg_v[b, pl.ds(q * 16, 16)] = g
                    idxs_v[b, pl.ds(q * 16, 16)] = sc
                    if gather_low:
                        idxn_v[b, pl.ds(q * 16, 16)] = lo
                gds.append(pltpu.async_copy(
                    src.at[idxg_v.at[b]], rows_v.at[b], gsem.at[b]))
                if gather_low:
                    hds.append((
                        pltpu.async_copy(ones_v, bdeg.at[idxs_v.at[b]],
                                         h1sem.at[b], add=True),
                        pltpu.async_copy(ones_v, ddeg.at[idxn_v.at[b]],
                                         h2sem.at[b], add=True)))
            for b in range(NBUF):
                gds[b].wait()
                sds.append(pltpu.async_copy(
                    rows_v.at[b], dst.at[idxs_v.at[b]], ssem.at[b],
                    add=True))
            for b in range(NBUF):
                sds[b].wait()
                if gather_low:
                    hds[b][0].wait()
                    hds[b][1].wait()
            return carry
        lax.fori_loop(0, NCH // NBUF, _p, 0)

    def _scale_in_place(degref, accref):
        pltpu.sync_copy(degref.at[pl.ds(s * SLICE, SLICE)], deg_v)

        def _inv(k, carry):
            v = deg_v[pl.ds(k * 16, 16)]
            deg_v[pl.ds(k * 16, 16)] = jnp.where(v > 0.0, 1.0 / v, 0.0)
            return carry
        lax.fori_loop(0, SLICE // 16, _inv, 0)

        def _u(u, ucarry):
            base = s * SLICE + u * SUB
            pltpu.sync_copy(accref.at[pl.ds(base, SUB)], big_v)

            def _row16(k, carry):
                vinv = deg_v[pl.ds(u * SUB + k * 16, 16)]
                for i in range(16):
                    r = k * 16 + i
                    inv = vinv[i]
                    for q in range(HALF // 16):
                        big_v[r, pl.ds(q * 16, 16)] = (
                            big_v[r, pl.ds(q * 16, 16)] * inv)
                return carry
            lax.fori_loop(0, SUB // 16, _row16, 0)
            pltpu.sync_copy(big_v, accref.at[pl.ds(base, SUB)])
            return ucarry
        lax.fori_loop(0, SLICE // SUB, _u, 0)

    _stream_pass(xwcat_hbm, agg, gather_low=True)
    plsc.subcore_barrier()

    _scale_in_place(bdeg, agg)
    plsc.subcore_barrier()

    _stream_pass(agg, nagg, gather_low=False)
    plsc.subcore_barrier()

    pltpu.sync_copy(nagg.at[pl.ds(s * SLICE, SLICE)],
                    out_hbm.at[pl.ds(coff + s * SLICE, SLICE)])
    pltpu.sync_copy(ddeg.at[pl.ds(s * SLICE, SLICE)],
                    ddeg_hbm.at[pl.ds(coff + s * SLICE, SLICE)])


def _build_sc_kernel(agg_rows):
    return functools.partial(
        pl.kernel,
        out_type=(
            jax.ShapeDtypeStruct((NC * NP, HALF), jnp.float32),
            jax.ShapeDtypeStruct((NC * NP,), jnp.float32),
        ),
        mesh=plsc.VectorSubcoreMesh(
            core_axis_name="c", subcore_axis_name="s",
            num_cores=NC, num_subcores=NS),
        compiler_params=pltpu.CompilerParams(use_tc_tiling_on_sc=False),
        scratch_types=[
            pltpu.VMEM((NCH, CH), jnp.int32),
            pltpu.VMEM((NBUF, CH, HALF), jnp.float32),
            pltpu.VMEM((SUB, HALF), jnp.float32),
            pltpu.VMEM((SLICE,), jnp.float32),
            pltpu.VMEM((CH,), jnp.float32),
            pltpu.VMEM((NBUF, CH), jnp.int32),
            pltpu.VMEM((NBUF, CH), jnp.int32),
            pltpu.VMEM((NBUF, CH), jnp.int32),
            pltpu.VMEM_SHARED((agg_rows, HALF), jnp.float32),
            pltpu.VMEM_SHARED((agg_rows, HALF), jnp.float32),
            pltpu.VMEM_SHARED((NP,), jnp.float32),
            pltpu.VMEM_SHARED((NP,), jnp.float32),
            pltpu.SemaphoreType.DMA((NBUF,)),
            pltpu.SemaphoreType.DMA((NBUF,)),
            pltpu.SemaphoreType.DMA((NBUF,)),
            pltpu.SemaphoreType.DMA((NBUF,)),
        ],
    )(_sc_body)


_sc_kernel = _build_sc_kernel(NP)


def kernel(x, hyperedge_index, hyperedge_attr, W, b, gamma, beta):
    del hyperedge_attr
    del b
    node_idx = hyperedge_index[0]
    edge_idx = hyperedge_index[1]

    x_pad = jnp.pad(x, ((0, NP - N), (0, 0)))
    pad_bins = (N + (jnp.arange(NNZ_PAD - NNZ, dtype=jnp.int32)
                     % NPAD_BINS)).astype(jnp.int32)
    nidx_p = jnp.concatenate([node_idx, pad_bins])
    eidx_p = jnp.concatenate([edge_idx, pad_bins])
    pidx = (nidx_p | (eidx_p << 16)).reshape(NS, NCH, CH)

    xw_cat = _matmul(x_pad, W).reshape(NC * NP, HALF)
    out_r, dd = _sc_kernel(xw_cat, pidx)
    return _batchnorm(out_r.reshape(NC, NP, HALF),
                      dd[:NP].reshape(NP, 1), gamma, beta)
